# baseline (speedup 1.0000x reference)
import functools

import jax
import jax.numpy as jnp
from jax import lax
from jax.experimental import pallas as pl
from jax.experimental.pallas import tpu as pltpu
from jax.experimental.pallas import tpu_sc as plsc


def _pass1_body(adj_ref, x_ref, w1_ref, b1_ref, w0_ref, u_ref, vh_ref,
                hf_ref, s_ref):
    @pl.when(pl.program_id(0) == 0)
    def _():
        s_ref[...] = jnp.dot(x_ref[...], w1_ref[...],
                             preferred_element_type=jnp.float32) + b1_ref[...]

    h = jnp.dot(adj_ref[...], s_ref[...], preferred_element_type=jnp.float32)
    h = jnp.maximum(h, 0.0)
    hdim = s_ref.shape[1]
    w0a = w0_ref[:hdim, :]
    w0b = w0_ref[hdim:, :]
    u = jnp.dot(h, w0a, preferred_element_type=jnp.float32)
    v = jnp.dot(h, w0b, preferred_element_type=jnp.float32)
    u_ref[...] = jnp.exp(jnp.minimum(-u, 60.0))
    ev = jnp.exp(jnp.minimum(-v, 60.0))
    eb = lax.shift_right_logical(
        lax.bitcast_convert_type(ev, jnp.int32) + 0x8000, 16)
    hb = (lax.bitcast_convert_type(h, jnp.int32) + 0x8000) & ~0xFFFF
    vh_ref[...] = lax.bitcast_convert_type(hb | eb, jnp.float32)
    hf_ref[...] = h


def _pass2_body(adj_ref, hf_ref, agg_ref, w2_ref, b2_ref, o_ref, s2_ref):
    @pl.when(pl.program_id(0) == 0)
    def _():
        h_new = hf_ref[...] + agg_ref[...]
        s2_ref[...] = jnp.dot(h_new, w2_ref[...],
                              preferred_element_type=jnp.float32) + b2_ref[...]

    o = jnp.dot(adj_ref[...], s2_ref[...], preferred_element_type=jnp.float32)
    m = jnp.max(o, axis=1, keepdims=True)
    e = jnp.exp(o - m)
    lse = jnp.log(jnp.sum(e, axis=1, keepdims=True))
    o_ref[...] = o - m - lse


_LANES = 16
_BATCH = 4
_NBUF = 2


def _make_sc_mask(n_pad, k_deg, hdim, n_workers):
    per_w = n_pad // n_workers
    n_rounds = per_w // _BATCH
    assert n_rounds % _NBUF == 0
    mesh = plsc.VectorSubcoreMesh(core_axis_name="c", subcore_axis_name="s")
    n_chunks = hdim // _LANES
    bk = _BATCH * k_deg

    @functools.partial(
        pl.kernel, mesh=mesh,
        out_type=jax.ShapeDtypeStruct((n_pad, hdim), jnp.float32),
        scratch_types=[
            pltpu.VMEM((per_w * k_deg,), jnp.int32),
            pltpu.VMEM_SHARED((n_pad, hdim), jnp.float32),
        ] + [pltpu.VMEM((bk, hdim), jnp.float32) for _ in range(_NBUF)]
          + [pltpu.VMEM((_BATCH, hdim), jnp.float32) for _ in range(2 * _NBUF)]
          + [pltpu.SemaphoreType.DMA for _ in range(3 * _NBUF)],
    )
    def sc_mask(nbr_hbm, u_hbm, vh_hbm, out_hbm, idx_v, table_sp, *rest):
        gbufs = rest[:_NBUF]
        ubufs = rest[_NBUF:2 * _NBUF]
        abufs = rest[2 * _NBUF:3 * _NBUF]
        gsems = rest[3 * _NBUF:4 * _NBUF]
        usems = rest[4 * _NBUF:5 * _NBUF]
        asems = rest[5 * _NBUF:6 * _NBUF]
        nc = lax.axis_size("c")
        ns = lax.axis_size("s")
        sid = lax.axis_index("s")
        wid = sid * nc + lax.axis_index("c")
        wbase = wid * per_w

        rows_per_tile = n_pad // ns
        pltpu.sync_copy(vh_hbm.at[pl.ds(sid * rows_per_tile, rows_per_tile)],
                        table_sp.at[pl.ds(sid * rows_per_tile, rows_per_tile)])
        pltpu.sync_copy(nbr_hbm.at[pl.ds(wbase * k_deg, per_w * k_deg)],
                        idx_v)
        plsc.subcore_barrier()

        def start(g, b):
            pltpu.make_async_copy(
                table_sp.at[idx_v.at[pl.ds(g * bk, bk)]], gbufs[b],
                gsems[b]).start()
            pltpu.make_async_copy(
                u_hbm.at[pl.ds(wbase + g * _BATCH, _BATCH)], ubufs[b],
                usems[b]).start()

        start(0, 0)

        def outer(gg, _):
            for b in range(_NBUF):
                g = gg * _NBUF + b

                @pl.when(g + 1 < n_rounds)
                def _(g=g, b=b):
                    start(g + 1, 1 - b)

                pltpu.make_async_copy(
                    table_sp.at[idx_v.at[pl.ds(0, bk)]], gbufs[b],
                    gsems[b]).wait()
                pltpu.make_async_copy(
                    u_hbm.at[pl.ds(0, _BATCH)], ubufs[b], usems[b]).wait()

                @pl.when(g >= _NBUF)
                def _(b=b):
                    pltpu.make_async_copy(
                        abufs[b], out_hbm.at[pl.ds(0, _BATCH)],
                        asems[b]).wait()

                rows_v = gbufs[b]
                u_v = ubufs[b]
                acc_v = abufs[b]

                def node_body(n, _, rows_v=rows_v, u_v=u_v, acc_v=acc_v):
                    magic = jnp.full((_LANES,), 0x7EF127EA, jnp.int32)
                    sh16 = jnp.full((_LANES,), 16, jnp.int32)
                    mhi = jnp.full((_LANES,), ~0xFFFF, jnp.int32)
                    for c in range(n_chunks):
                        euc = u_v[n, pl.ds(c * _LANES, _LANES)]
                        acc = jnp.zeros((_LANES,), jnp.float32)
                        for k in range(k_deg):
                            w = lax.bitcast_convert_type(
                                rows_v[n * k_deg + k,
                                       pl.ds(c * _LANES, _LANES)], jnp.int32)
                            evk = lax.bitcast_convert_type(
                                lax.shift_left(w, sh16), jnp.float32)
                            hh = lax.bitcast_convert_type(
                                w & mhi, jnp.float32)
                            e = jnp.minimum(euc * evk, 1e30)
                            d = 1.0 + e
                            r = lax.bitcast_convert_type(
                                magic - lax.bitcast_convert_type(d, jnp.int32),
                                jnp.float32)
                            r = r * (2.0 - d * r)
                            r = r * (2.0 - d * r)
                            acc = acc + hh * r
                        acc_v[n, pl.ds(c * _LANES, _LANES)] = acc
                    return 0

                lax.fori_loop(0, _BATCH, node_body, 0)
                pltpu.make_async_copy(
                    abufs[b],
                    out_hbm.at[pl.ds(wbase + g * _BATCH, _BATCH)],
                    asems[b]).start()
            return 0

        lax.fori_loop(0, n_rounds // _NBUF, outer, 0)
        for b in range(_NBUF):
            pltpu.make_async_copy(
                abufs[b], out_hbm.at[pl.ds(0, _BATCH)], asems[b]).wait()

    return sc_mask


def kernel(x, adj, neighbors, W1, b1, W0, W2, b2):
    n, nfeat = x.shape
    hdim = W1.shape[1]
    ncls = W2.shape[1]
    k_deg = neighbors.shape[1]

    row_blk = 400
    n_blocks = n // row_blk
    tc_params = pltpu.CompilerParams(vmem_limit_bytes=63 * 1024 * 1024)
    n_workers = 32
    gran = n_workers * _BATCH * _NBUF * 2
    n_pad = ((n + gran - 1) // gran) * gran

    eu, vh, hf = pl.pallas_call(
        _pass1_body,
        grid=(n_blocks,),
        in_specs=[
            pl.BlockSpec((row_blk, n), lambda i: (i, 0)),
            pl.BlockSpec((n, nfeat), lambda i: (0, 0)),
            pl.BlockSpec((nfeat, hdim), lambda i: (0, 0)),
            pl.BlockSpec((1, hdim), lambda i: (0, 0)),
            pl.BlockSpec((2 * hdim, hdim), lambda i: (0, 0)),
        ],
        out_specs=[
            pl.BlockSpec((row_blk, hdim), lambda i: (i, 0)),
            pl.BlockSpec((row_blk, hdim), lambda i: (i, 0)),
            pl.BlockSpec((row_blk, hdim), lambda i: (i, 0)),
        ],
        out_shape=[
            jax.ShapeDtypeStruct((n_pad, hdim), jnp.float32),
            jax.ShapeDtypeStruct((n_pad, hdim), jnp.float32),
            jax.ShapeDtypeStruct((n, hdim), jnp.float32),
        ],
        scratch_shapes=[pltpu.VMEM((n, hdim), jnp.float32)],
        compiler_params=tc_params,
    )(adj, x, W1, b1.reshape(1, hdim), W0)

    nbr_flat = jnp.concatenate(
        [neighbors.astype(jnp.int32).reshape(-1),
         jnp.zeros(((n_pad - n) * k_deg,), jnp.int32)])
    agg = _make_sc_mask(n_pad, k_deg, hdim, n_workers)(nbr_flat, eu, vh)

    out = pl.pallas_call(
        _pass2_body,
        grid=(n_blocks,),
        in_specs=[
            pl.BlockSpec((row_blk, n), lambda i: (i, 0)),
            pl.BlockSpec((n, hdim), lambda i: (0, 0)),
            pl.BlockSpec((n, hdim), lambda i: (0, 0)),
            pl.BlockSpec((hdim, ncls), lambda i: (0, 0)),
            pl.BlockSpec((1, ncls), lambda i: (0, 0)),
        ],
        out_specs=pl.BlockSpec((row_blk, ncls), lambda i: (i, 0)),
        out_shape=jax.ShapeDtypeStruct((n, ncls), jnp.float32),
        scratch_shapes=[pltpu.VMEM((n, ncls), jnp.float32)],
        compiler_params=tc_params,
    )(adj, hf, agg, W2, b2.reshape(1, ncls))

    return out


# speedup vs baseline: 1.0452x; 1.0452x over previous
import functools

import jax
import jax.numpy as jnp
from jax import lax
from jax.experimental import pallas as pl
from jax.experimental.pallas import tpu as pltpu
from jax.experimental.pallas import tpu_sc as plsc


def _pass1_body(adj_ref, x_ref, w1_ref, b1_ref, w0_ref, u_ref, vh_ref,
                hf_ref, s_ref):
    @pl.when(pl.program_id(0) == 0)
    def _():
        s_ref[...] = jnp.dot(x_ref[...], w1_ref[...],
                             preferred_element_type=jnp.float32) + b1_ref[...]

    h = jnp.dot(adj_ref[...], s_ref[...], preferred_element_type=jnp.float32)
    h = jnp.maximum(h, 0.0)
    hdim = s_ref.shape[1]
    w0a = w0_ref[:hdim, :]
    w0b = w0_ref[hdim:, :]
    u = jnp.dot(h, w0a, preferred_element_type=jnp.float32)
    v = jnp.dot(h, w0b, preferred_element_type=jnp.float32)
    u_ref[...] = jnp.exp(jnp.minimum(-u, 34.0))
    ev = jnp.exp(jnp.minimum(-v, 34.0))
    eb = lax.shift_right_logical(
        lax.bitcast_convert_type(ev, jnp.int32) + 0x8000, 16)
    hb = (lax.bitcast_convert_type(h, jnp.int32) + 0x8000) & ~0xFFFF
    vh_ref[...] = lax.bitcast_convert_type(hb | eb, jnp.float32)
    hf_ref[...] = h


def _pass2_body(adj_ref, hf_ref, agg_ref, w2_ref, b2_ref, o_ref, s2_ref):
    @pl.when(pl.program_id(0) == 0)
    def _():
        h_new = hf_ref[...] + agg_ref[...]
        s2_ref[...] = jnp.dot(h_new, w2_ref[...],
                              preferred_element_type=jnp.float32) + b2_ref[...]

    o = jnp.dot(adj_ref[...], s2_ref[...], preferred_element_type=jnp.float32)
    m = jnp.max(o, axis=1, keepdims=True)
    e = jnp.exp(o - m)
    lse = jnp.log(jnp.sum(e, axis=1, keepdims=True))
    o_ref[...] = o - m - lse


_LANES = 16
_BATCH = 4
_NBUF = 2


def _make_sc_mask(n_pad, k_deg, hdim, n_workers):
    per_w = n_pad // n_workers
    n_rounds = per_w // _BATCH
    assert n_rounds % _NBUF == 0
    mesh = plsc.VectorSubcoreMesh(core_axis_name="c", subcore_axis_name="s")
    n_chunks = hdim // _LANES
    bk = _BATCH * k_deg

    @functools.partial(
        pl.kernel, mesh=mesh,
        out_type=jax.ShapeDtypeStruct((n_pad, hdim), jnp.float32),
        scratch_types=[
            pltpu.VMEM((per_w * k_deg,), jnp.int32),
            pltpu.VMEM_SHARED((n_pad, hdim), jnp.float32),
        ] + [pltpu.VMEM((bk, hdim), jnp.float32) for _ in range(_NBUF)]
          + [pltpu.VMEM((_BATCH, hdim), jnp.float32) for _ in range(2 * _NBUF)]
          + [pltpu.SemaphoreType.DMA for _ in range(3 * _NBUF)],
    )
    def sc_mask(nbr_hbm, u_hbm, vh_hbm, out_hbm, idx_v, table_sp, *rest):
        gbufs = rest[:_NBUF]
        ubufs = rest[_NBUF:2 * _NBUF]
        abufs = rest[2 * _NBUF:3 * _NBUF]
        gsems = rest[3 * _NBUF:4 * _NBUF]
        usems = rest[4 * _NBUF:5 * _NBUF]
        asems = rest[5 * _NBUF:6 * _NBUF]
        nc = lax.axis_size("c")
        ns = lax.axis_size("s")
        sid = lax.axis_index("s")
        wid = sid * nc + lax.axis_index("c")
        wbase = wid * per_w

        rows_per_tile = n_pad // ns
        pltpu.sync_copy(vh_hbm.at[pl.ds(sid * rows_per_tile, rows_per_tile)],
                        table_sp.at[pl.ds(sid * rows_per_tile, rows_per_tile)])
        pltpu.sync_copy(nbr_hbm.at[pl.ds(wbase * k_deg, per_w * k_deg)],
                        idx_v)
        plsc.subcore_barrier()

        def start(g, b):
            pltpu.make_async_copy(
                table_sp.at[idx_v.at[pl.ds(g * bk, bk)]], gbufs[b],
                gsems[b]).start()
            pltpu.make_async_copy(
                u_hbm.at[pl.ds(wbase + g * _BATCH, _BATCH)], ubufs[b],
                usems[b]).start()

        start(0, 0)

        def outer(gg, _):
            for b in range(_NBUF):
                g = gg * _NBUF + b

                @pl.when(g + 1 < n_rounds)
                def _(g=g, b=b):
                    start(g + 1, 1 - b)

                pltpu.make_async_copy(
                    table_sp.at[idx_v.at[pl.ds(0, bk)]], gbufs[b],
                    gsems[b]).wait()
                pltpu.make_async_copy(
                    u_hbm.at[pl.ds(0, _BATCH)], ubufs[b], usems[b]).wait()

                @pl.when(g >= _NBUF)
                def _(b=b):
                    pltpu.make_async_copy(
                        abufs[b], out_hbm.at[pl.ds(0, _BATCH)],
                        asems[b]).wait()

                rows_v = gbufs[b]
                u_v = ubufs[b]
                acc_v = abufs[b]

                def node_body(n, _, rows_v=rows_v, u_v=u_v, acc_v=acc_v):
                    magic = jnp.full((_LANES,), 0x7EF127EA, jnp.int32)
                    sh16 = jnp.full((_LANES,), 16, jnp.int32)
                    mhi = jnp.full((_LANES,), ~0xFFFF, jnp.int32)
                    for c in range(n_chunks):
                        euc = u_v[n, pl.ds(c * _LANES, _LANES)]
                        acc = jnp.zeros((_LANES,), jnp.float32)
                        for k in range(k_deg):
                            w = lax.bitcast_convert_type(
                                rows_v[n * k_deg + k,
                                       pl.ds(c * _LANES, _LANES)], jnp.int32)
                            evk = lax.bitcast_convert_type(
                                lax.shift_left(w, sh16), jnp.float32)
                            hh = lax.bitcast_convert_type(
                                w & mhi, jnp.float32)
                            d = 1.0 + euc * evk
                            r = lax.bitcast_convert_type(
                                magic - lax.bitcast_convert_type(d, jnp.int32),
                                jnp.float32)
                            r = r * (2.0 - d * r)
                            acc = acc + hh * r
                        acc_v[n, pl.ds(c * _LANES, _LANES)] = acc
                    return 0

                lax.fori_loop(0, _BATCH, node_body, 0)
                pltpu.make_async_copy(
                    abufs[b],
                    out_hbm.at[pl.ds(wbase + g * _BATCH, _BATCH)],
                    asems[b]).start()
            return 0

        lax.fori_loop(0, n_rounds // _NBUF, outer, 0)
        for b in range(_NBUF):
            pltpu.make_async_copy(
                abufs[b], out_hbm.at[pl.ds(0, _BATCH)], asems[b]).wait()

    return sc_mask


def kernel(x, adj, neighbors, W1, b1, W0, W2, b2):
    n, nfeat = x.shape
    hdim = W1.shape[1]
    ncls = W2.shape[1]
    k_deg = neighbors.shape[1]

    row_blk = 400
    n_blocks = n // row_blk
    tc_params = pltpu.CompilerParams(vmem_limit_bytes=63 * 1024 * 1024)
    n_workers = 32
    gran = n_workers * _BATCH * _NBUF * 2
    n_pad = ((n + gran - 1) // gran) * gran

    eu, vh, hf = pl.pallas_call(
        _pass1_body,
        grid=(n_blocks,),
        in_specs=[
            pl.BlockSpec((row_blk, n), lambda i: (i, 0)),
            pl.BlockSpec((n, nfeat), lambda i: (0, 0)),
            pl.BlockSpec((nfeat, hdim), lambda i: (0, 0)),
            pl.BlockSpec((1, hdim), lambda i: (0, 0)),
            pl.BlockSpec((2 * hdim, hdim), lambda i: (0, 0)),
        ],
        out_specs=[
            pl.BlockSpec((row_blk, hdim), lambda i: (i, 0)),
            pl.BlockSpec((row_blk, hdim), lambda i: (i, 0)),
            pl.BlockSpec((row_blk, hdim), lambda i: (i, 0)),
        ],
        out_shape=[
            jax.ShapeDtypeStruct((n_pad, hdim), jnp.float32),
            jax.ShapeDtypeStruct((n_pad, hdim), jnp.float32),
            jax.ShapeDtypeStruct((n, hdim), jnp.float32),
        ],
        scratch_shapes=[pltpu.VMEM((n, hdim), jnp.float32)],
        compiler_params=tc_params,
    )(adj, x, W1, b1.reshape(1, hdim), W0)

    nbr_flat = jnp.concatenate(
        [neighbors.astype(jnp.int32).reshape(-1),
         jnp.zeros(((n_pad - n) * k_deg,), jnp.int32)])
    agg = _make_sc_mask(n_pad, k_deg, hdim, n_workers)(nbr_flat, eu, vh)

    out = pl.pallas_call(
        _pass2_body,
        grid=(n_blocks,),
        in_specs=[
            pl.BlockSpec((row_blk, n), lambda i: (i, 0)),
            pl.BlockSpec((n, hdim), lambda i: (0, 0)),
            pl.BlockSpec((n, hdim), lambda i: (0, 0)),
            pl.BlockSpec((hdim, ncls), lambda i: (0, 0)),
            pl.BlockSpec((1, ncls), lambda i: (0, 0)),
        ],
        out_specs=pl.BlockSpec((row_blk, ncls), lambda i: (i, 0)),
        out_shape=jax.ShapeDtypeStruct((n, ncls), jnp.float32),
        scratch_shapes=[pltpu.VMEM((n, ncls), jnp.float32)],
        compiler_params=tc_params,
    )(adj, hf, agg, W2, b2.reshape(1, ncls))

    return out
